# baseline (speedup 1.0000x reference)
import functools

import jax
import jax.numpy as jnp
from jax import lax
from jax.experimental import pallas as pl
from jax.experimental.pallas import tpu as pltpu
from jax.experimental.pallas import tpu_sc as plsc

B = 16384
DU = 64
DI = 32
FU = 128
FI = 128
FC = 128
NUM_TASKS = 4
CROSS_OUT = 128

NC = 2
NS = 16
NW = NC * NS
BPW = B // NW

BB = 2048


IDX_CHUNK = 128
N_CHUNKS = BPW // IDX_CHUNK


def _sc_gather_body(user_tab, uid, item_tab, iid, ue_out, ie_out,
                    uidx_v, urows_v, iidx_v, irows_v, usem, isem):
    wid = lax.axis_index("s") * NC + lax.axis_index("c")
    base = wid * BPW
    pltpu.sync_copy(uid.at[pl.ds(base, BPW)], uidx_v)
    pltpu.sync_copy(iid.at[pl.ds(base, BPW)], iidx_v)
    copies = []
    for j in range(N_CHUNKS):
        s = pl.ds(j * IDX_CHUNK, IDX_CHUNK)
        copies.append(pltpu.async_copy(
            user_tab.at[uidx_v.at[s]], urows_v.at[s], usem))
        copies.append(pltpu.async_copy(
            item_tab.at[iidx_v.at[s]], irows_v.at[s], isem))
    for c in copies:
        c.wait()
    pltpu.sync_copy(urows_v, ue_out.at[pl.ds(base, BPW)])
    pltpu.sync_copy(irows_v, ie_out.at[pl.ds(base, BPW)])


_sc_gather = pl.kernel(
    _sc_gather_body,
    out_type=(
        jax.ShapeDtypeStruct((B, DU), jnp.float32),
        jax.ShapeDtypeStruct((B, DI), jnp.float32),
    ),
    mesh=plsc.VectorSubcoreMesh(core_axis_name="c", subcore_axis_name="s"),
    compiler_params=pltpu.CompilerParams(use_tc_tiling_on_sc=False),
    scratch_types=[
        pltpu.VMEM((BPW,), jnp.int32),
        pltpu.VMEM((BPW, DU), jnp.float32),
        pltpu.VMEM((BPW,), jnp.int32),
        pltpu.VMEM((BPW, DI), jnp.float32),
        pltpu.SemaphoreType.DMA,
        pltpu.SemaphoreType.DMA,
    ],
)


def _dense_body(uf, itf, cf, ue, ie, wu, wi, wc, wt, bu, bi, bc, bt, out):
    uft = jnp.dot(uf[...], wu[...], preferred_element_type=jnp.float32) + bu[...]
    ift = jnp.dot(itf[...], wi[...], preferred_element_type=jnp.float32) + bi[...]
    cft = jnp.dot(cf[...], wc[...], preferred_element_type=jnp.float32) + bc[...]
    wt_all = wt[...]
    acc = jnp.dot(ue[...], wt_all[0:DU, :], preferred_element_type=jnp.float32)
    acc += jnp.dot(uft, wt_all[DU:DU + DU, :], preferred_element_type=jnp.float32)
    acc += jnp.dot(ie[...], wt_all[2 * DU:2 * DU + DI, :],
                   preferred_element_type=jnp.float32)
    acc += jnp.dot(ift, wt_all[2 * DU + DI:2 * DU + 2 * DI, :],
                   preferred_element_type=jnp.float32)
    acc += jnp.dot(cft, wt_all[2 * DU + 2 * DI:, :],
                   preferred_element_type=jnp.float32)
    out[...] = acc + bt[...]


def _dense_call(uf, itf, cf, ue, ie, wu, wi, wc, wt, bu, bi, bc, bt):
    grid = (B // BB,)
    row_blk = lambda w: pl.BlockSpec((BB, w), lambda i: (i, 0))
    full = lambda a: pl.BlockSpec(a.shape, lambda i: tuple(0 for _ in a.shape))
    return pl.pallas_call(
        _dense_body,
        grid=grid,
        in_specs=[
            row_blk(FU), row_blk(FI), row_blk(FC), row_blk(DU), row_blk(DI),
            full(wu), full(wi), full(wc), full(wt),
            full(bu), full(bi), full(bc), full(bt),
        ],
        out_specs=pl.BlockSpec((BB, NUM_TASKS), lambda i: (i, 0)),
        out_shape=jax.ShapeDtypeStruct((B, NUM_TASKS), jnp.float32),
    )(uf, itf, cf, ue, ie, wu, wi, wc, wt, bu, bi, bc, bt)


def kernel(user_id, user_features, item_id, item_features, cross_features,
           position, user_table, item_table, W_u, b_u, W_i, b_i, W_c, b_c,
           W_t, b_t):
    ue, ie = _sc_gather(user_table, user_id, item_table, item_id)
    return _dense_call(
        user_features, item_features, cross_features, ue, ie,
        W_u, W_i, W_c, W_t,
        b_u.reshape(1, DU), b_i.reshape(1, DI), b_c.reshape(1, CROSS_OUT),
        b_t.reshape(1, NUM_TASKS))
